# baseline (speedup 1.0000x reference)
import functools

import jax
import jax.numpy as jnp
from jax import lax
from jax.experimental import pallas as pl
from jax.experimental.pallas import tpu as pltpu
from jax.experimental.pallas import tpu_sc as plsc

BATCH = 16384
EMBED_K = 32
LANES = 16
NUM_WORKERS = 32
ROWS_PER_W = BATCH // NUM_WORKERS
CHUNK = 128
NCHUNK = ROWS_PER_W // CHUNK
GROUPS = ROWS_PER_W // LANES


def _sc_body(uidx_hbm, iidx_hbm, w_hbm, h_hbm, out_hbm,
             idx_u, idx_i, u_rows, v_rows, out_v, sem):
    nc = 2
    wid = lax.axis_index("s") * nc + lax.axis_index("c")

    pltpu.sync_copy(uidx_hbm.at[pl.ds(wid * NCHUNK, NCHUNK)], idx_u)
    pltpu.sync_copy(iidx_hbm.at[pl.ds(wid * NCHUNK, NCHUNK)], idx_i)

    copies = []
    for j in range(NCHUNK):
        copies.append(pltpu.async_copy(
            w_hbm.at[idx_u.at[j]], u_rows.at[pl.ds(j * CHUNK, CHUNK)], sem))
        copies.append(pltpu.async_copy(
            h_hbm.at[idx_i.at[j]], v_rows.at[pl.ds(j * CHUNK, CHUNK)], sem))
    for c in copies:
        c.wait()

    lane = lax.iota(jnp.int32, LANES)

    def group_body(g, _):
        acc = jnp.zeros((LANES,), jnp.float32)
        for rr in range(LANES):
            r = g * LANES + rr
            u0 = u_rows[r, pl.ds(0, LANES)]
            u1 = u_rows[r, pl.ds(LANES, LANES)]
            v0 = v_rows[r, pl.ds(0, LANES)]
            v1 = v_rows[r, pl.ds(LANES, LANES)]
            s = jnp.sum(u0 * v0 + u1 * v1)
            acc = jnp.where(lane == rr, s, acc)
        out_v[pl.ds(g * LANES, LANES)] = acc
        return _

    lax.fori_loop(0, GROUPS, group_body, 0)

    pltpu.sync_copy(out_v, out_hbm.at[pl.ds(wid * ROWS_PER_W, ROWS_PER_W)])


@jax.jit
def kernel(x, W, H):
    uidx = x[:, 0].reshape(NUM_WORKERS * NCHUNK, CHUNK).astype(jnp.int32)
    iidx = x[:, 1].reshape(NUM_WORKERS * NCHUNK, CHUNK).astype(jnp.int32)

    mesh = plsc.VectorSubcoreMesh(core_axis_name="c", subcore_axis_name="s")
    run = functools.partial(
        pl.kernel,
        out_type=jax.ShapeDtypeStruct((BATCH,), jnp.float32),
        mesh=mesh,
        compiler_params=pltpu.CompilerParams(
            needs_layout_passes=False, use_tc_tiling_on_sc=False),
        scratch_types=[
            pltpu.VMEM((NCHUNK, CHUNK), jnp.int32),
            pltpu.VMEM((NCHUNK, CHUNK), jnp.int32),
            pltpu.VMEM((ROWS_PER_W, EMBED_K), jnp.float32),
            pltpu.VMEM((ROWS_PER_W, EMBED_K), jnp.float32),
            pltpu.VMEM((ROWS_PER_W,), jnp.float32),
            pltpu.SemaphoreType.DMA,
        ],
    )(_sc_body)
    return run(uidx, iidx, W, H)


# speedup vs baseline: 1.4995x; 1.4995x over previous
import functools

import jax
import jax.numpy as jnp
from jax import lax
from jax.experimental import pallas as pl
from jax.experimental.pallas import tpu as pltpu
from jax.experimental.pallas import tpu_sc as plsc

BATCH = 16384
EMBED_K = 32
LANES = 16
NUM_WORKERS = 32
ROWS_PER_W = BATCH // NUM_WORKERS
CH = 128
NCH = ROWS_PER_W // CH
GROUPS_PER_CH = CH // LANES


def _sc_body(uidx_hbm, iidx_hbm, w_hbm, h_hbm, out_hbm,
             uid_v, iid_v, u0b, u1b, v0b, v1b, drain, out_v,
             sem0, sem1):
    nc = 2
    wid = lax.axis_index("s") * nc + lax.axis_index("c")
    base = wid * ROWS_PER_W

    pltpu.sync_copy(uidx_hbm.at[pl.ds(base, ROWS_PER_W)], uid_v)
    pltpu.sync_copy(iidx_hbm.at[pl.ds(base, ROWS_PER_W)], iid_v)

    ubufs = (u0b, u1b)
    vbufs = (v0b, v1b)
    sems = (sem0, sem1)

    def fire(c, ubuf, vbuf, sem):
        def body(g, _):
            uvec = uid_v[pl.ds(c * CH + g * LANES, LANES)]
            ivec = iid_v[pl.ds(c * CH + g * LANES, LANES)]
            for rr in range(LANES):
                i = g * LANES + rr
                pltpu.async_copy(
                    w_hbm.at[uvec[rr]], ubuf.at[i, pl.ds(0, EMBED_K)], sem)
                pltpu.async_copy(
                    h_hbm.at[ivec[rr]], vbuf.at[i, pl.ds(0, EMBED_K)], sem)
            return _
        lax.fori_loop(0, GROUPS_PER_CH, body, 0)

    def drain_chunk(sem):
        pltpu.make_async_copy(
            uidx_hbm.at[pl.ds(0, 2 * CH * EMBED_K)], drain, sem).wait()

    lane = lax.iota(jnp.int32, LANES)

    def compute(c, ubuf, vbuf):
        def group_body(g, _):
            acc = jnp.zeros((LANES,), jnp.float32)
            for rr in range(LANES):
                r = g * LANES + rr
                w0 = ubuf[r, pl.ds(0, LANES)]
                w1 = ubuf[r, pl.ds(LANES, LANES)]
                h0 = vbuf[r, pl.ds(0, LANES)]
                h1 = vbuf[r, pl.ds(LANES, LANES)]
                s = jnp.sum(w0 * h0 + w1 * h1)
                acc = jnp.where(lane == rr, s, acc)
            out_v[pl.ds(c * CH + g * LANES, LANES)] = acc
            return _
        lax.fori_loop(0, GROUPS_PER_CH, group_body, 0)

    fire(0, ubufs[0], vbufs[0], sems[0])
    for c in range(NCH):
        if c + 1 < NCH:
            p = (c + 1) % 2
            fire(c + 1, ubufs[p], vbufs[p], sems[p])
        drain_chunk(sems[c % 2])
        compute(c, ubufs[c % 2], vbufs[c % 2])

    pltpu.sync_copy(out_v, out_hbm.at[pl.ds(base, ROWS_PER_W)])


@jax.jit
def kernel(x, W, H):
    uidx = x[:, 0].astype(jnp.int32)
    iidx = x[:, 1].astype(jnp.int32)

    mesh = plsc.VectorSubcoreMesh(core_axis_name="c", subcore_axis_name="s")
    run = functools.partial(
        pl.kernel,
        out_type=jax.ShapeDtypeStruct((BATCH,), jnp.float32),
        mesh=mesh,
        compiler_params=pltpu.CompilerParams(
            needs_layout_passes=False, use_tc_tiling_on_sc=True),
        scratch_types=[
            pltpu.VMEM((ROWS_PER_W,), jnp.int32),
            pltpu.VMEM((ROWS_PER_W,), jnp.int32),
            pltpu.VMEM((CH, 128), jnp.float32),
            pltpu.VMEM((CH, 128), jnp.float32),
            pltpu.VMEM((CH, 128), jnp.float32),
            pltpu.VMEM((CH, 128), jnp.float32),
            pltpu.VMEM((2 * CH * EMBED_K,), jnp.int32),
            pltpu.VMEM((ROWS_PER_W,), jnp.float32),
            pltpu.SemaphoreType.DMA,
            pltpu.SemaphoreType.DMA,
        ],
    )(_sc_body)
    return run(uidx, iidx, W, H)
